# baseline (speedup 1.0000x reference)
import functools
import jax
import jax.numpy as jnp
from jax import lax
from jax.experimental import pallas as pl
from jax.experimental.pallas import tpu as pltpu, tpu_sc as plsc

_BATCH = 16384
_D = 1000
_NW = 32
_BPW = _BATCH // _NW
_C = 64
_NCHUNK = _BPW // _C
_L = 16


def _sc_body(x_hbm, out_hbm, idx_v, buf0, buf1, sem0, sem1):
    wid = lax.axis_index("s") * 2 + lax.axis_index("c")
    base = wid * _BPW
    pltpu.sync_copy(x_hbm.at[pl.ds(base, _BPW)], idx_v)

    zeros = jnp.zeros((_L,), jnp.float32)
    ones = jnp.ones((_L,), jnp.float32)
    lane = lax.iota(jnp.int32, _L)

    bufs = (buf0, buf1)
    sems = (sem0, sem1)

    def _zero_buf(buf):
        def body(i, _):
            buf[pl.ds(i * _L, _L)] = zeros
            return 0
        lax.fori_loop(0, (_C * _D) // _L, body, 0)

    _zero_buf(buf0)
    _zero_buf(buf1)

    def _positions(c, g):
        iv = idx_v[pl.ds(c * _C + g * _L, _L)]
        return iv + (lane + g * _L) * _D

    copies = [None, None]
    for c in range(_NCHUNK):
        b = c % 2
        buf = bufs[b]
        if c >= 2:
            copies[b].wait()
            for g in range(_C // _L):
                plsc.store_scatter(buf, [_positions(c - 2, g)], zeros)
        for g in range(_C // _L):
            plsc.store_scatter(buf, [_positions(c, g)], ones)
        cp = pltpu.make_async_copy(
            buf, out_hbm.at[pl.ds(base * _D + c * _C * _D, _C * _D)], sems[b])
        cp.start()
        copies[b] = cp
    copies[(_NCHUNK - 2) % 2].wait()
    copies[(_NCHUNK - 1) % 2].wait()


@functools.partial(jax.jit, donate_argnums=())
def _sc_onehot(x):
    mesh = plsc.VectorSubcoreMesh(core_axis_name="c", subcore_axis_name="s")
    f = pl.kernel(
        _sc_body,
        mesh=mesh,
        compiler_params=pltpu.CompilerParams(needs_layout_passes=False),
        out_type=jax.ShapeDtypeStruct((_BATCH * _D,), jnp.float32),
        scratch_types=[
            pltpu.VMEM((_BPW,), jnp.int32),
            pltpu.VMEM((_C * _D,), jnp.float32),
            pltpu.VMEM((_C * _D,), jnp.float32),
            pltpu.SemaphoreType.DMA,
            pltpu.SemaphoreType.DMA,
        ],
    )
    return f(x)


def kernel(x, table):
    del table
    return _sc_onehot(x).reshape(_BATCH, _D)


# speedup vs baseline: 1.9262x; 1.9262x over previous
import functools
import jax
import jax.numpy as jnp
from jax import lax
from jax.experimental import pallas as pl
from jax.experimental.pallas import tpu as pltpu, tpu_sc as plsc

_BATCH = 16384
_D = 1000
_NW = 32
_BPW = _BATCH // _NW
_C = 32
_NCHUNK = _BPW // _C
_L = 16


def _sc_body(x_hbm, out_hbm, idx_v, buf0, buf1, sem0, sem1):
    wid = lax.axis_index("s") * 2 + lax.axis_index("c")
    base = wid * _BPW
    pltpu.sync_copy(x_hbm.at[pl.ds(base, _BPW)], idx_v)

    zeros = jnp.zeros((_L,), jnp.float32)
    ones = jnp.ones((_L,), jnp.float32)
    lane = lax.iota(jnp.int32, _L)

    bufs = (buf0, buf1)
    sems = (sem0, sem1)

    col_offs = [16 * j for j in range(62)] + [984]

    def _zero_buf(buf):
        def body(r, _):
            for off in col_offs:
                buf[r, pl.ds(off, _L)] = zeros
            return 0
        lax.fori_loop(0, _C, body, 0)

    _zero_buf(buf0)
    _zero_buf(buf1)

    def _scatter(buf, c, vals):
        for g in range(_C // _L):
            iv = idx_v[pl.ds(c * _C + g * _L, _L)]
            plsc.store_scatter(buf, [lane + g * _L, iv], vals)

    copies = [None, None]
    for c in range(_NCHUNK):
        b = c % 2
        buf = bufs[b]
        if c >= 2:
            copies[b].wait()
            _scatter(buf, c - 2, zeros)
        _scatter(buf, c, ones)
        cp = pltpu.make_async_copy(
            buf, out_hbm.at[pl.ds(base + c * _C, _C)], sems[b])
        cp.start()
        copies[b] = cp
    copies[(_NCHUNK - 2) % 2].wait()
    copies[(_NCHUNK - 1) % 2].wait()


@jax.jit
def _sc_onehot(x):
    mesh = plsc.VectorSubcoreMesh(core_axis_name="c", subcore_axis_name="s")
    f = pl.kernel(
        _sc_body,
        mesh=mesh,
        compiler_params=pltpu.CompilerParams(
            needs_layout_passes=False,
            use_tc_tiling_on_sc=True,
        ),
        out_type=jax.ShapeDtypeStruct((_BATCH, _D), jnp.float32),
        scratch_types=[
            pltpu.VMEM((_BPW,), jnp.int32),
            pltpu.VMEM((_C, _D), jnp.float32),
            pltpu.VMEM((_C, _D), jnp.float32),
            pltpu.SemaphoreType.DMA,
            pltpu.SemaphoreType.DMA,
        ],
    )
    return f(x)


def kernel(x, table):
    del table
    return _sc_onehot(x)


# speedup vs baseline: 8.8721x; 4.6060x over previous
import jax
import jax.numpy as jnp
from jax.experimental import pallas as pl

_BATCH = 16384
_NUM_CLASS = 1000
_BN = 2048


def _onehot_t_block(x_ref, o_ref):
    xb = x_ref[0, 0, :]
    rows = jax.lax.broadcasted_iota(jnp.int32, o_ref.shape, 0)
    o_ref[...] = jnp.where(rows == xb[None, :], 1.0, 0.0).astype(o_ref.dtype)


def kernel(x, table):
    del table
    grid = _BATCH // _BN
    x3 = x.reshape(grid, 1, _BN)
    out_t = pl.pallas_call(
        _onehot_t_block,
        grid=(grid,),
        in_specs=[pl.BlockSpec((1, 1, _BN), lambda i: (i, 0, 0))],
        out_specs=pl.BlockSpec((_NUM_CLASS, _BN), lambda i: (0, i)),
        out_shape=jax.ShapeDtypeStruct((_NUM_CLASS, _BATCH), jnp.float32),
    )(x3)
    return out_t.T
